# Initial kernel scaffold; baseline (speedup 1.0000x reference)
import jax
import jax.numpy as jnp
from jax.experimental import pallas as pl


def kernel(x, W_router, w1, w2, w3):
    raise NotImplementedError("write your pallas kernel here")


# speedup vs baseline: 1.3034x; 1.3034x over previous
import functools

import jax
import jax.numpy as jnp
from jax import lax
from jax.experimental import pallas as pl
from jax.experimental.pallas import tpu as pltpu

HIDDEN = 1024
FFN = 4096
E = 8
EP = 128
BF = 512
NF = FFN // BF


def _router_body(x_ref, wr_ref, dw_ref):
    x = x_ref[...]
    wr = wr_ref[...]
    logits = lax.dot_general(x, wr, (((1,), (1,)), ((), ())),
                             preferred_element_type=jnp.float32)
    col = lax.broadcasted_iota(jnp.int32, logits.shape, 1)
    neg = jnp.float32(-1e30)
    logits = jnp.where(col < E, logits, neg)
    m1 = jnp.max(logits, axis=1, keepdims=True)
    a1 = jnp.min(jnp.where(logits == m1, col, EP), axis=1, keepdims=True)
    l2 = jnp.where(col == a1, neg, logits)
    m2 = jnp.max(l2, axis=1, keepdims=True)
    a2 = jnp.min(jnp.where(l2 == m2, col, EP), axis=1, keepdims=True)
    wtop = 1.0 / (1.0 + jnp.exp(m2 - m1))
    dw = jnp.where(col == a1, wtop, 0.0) + jnp.where(col == a2, 1.0 - wtop, 0.0)
    dw_ref[...] = dw


def _ffn_body(dw_ref, x_ref, w1_ref, w3_ref, w2_ref, out_ref):
    e = pl.program_id(0)
    f = pl.program_id(1)
    x = x_ref[...]
    h1 = jnp.dot(x, w1_ref[0], preferred_element_type=jnp.float32)
    h3 = jnp.dot(x, w3_ref[0], preferred_element_type=jnp.float32)
    h = (h1 * jax.nn.sigmoid(h1)) * h3
    col = lax.broadcasted_iota(jnp.int32, dw_ref.shape, 1)
    dw = jnp.sum(jnp.where(col == e, dw_ref[...], 0.0), axis=1, keepdims=True)
    delta = jnp.dot(h * dw, w2_ref[0], preferred_element_type=jnp.float32)

    @pl.when(jnp.logical_and(e == 0, f == 0))
    def _():
        out_ref[...] = jnp.zeros_like(out_ref)

    out_ref[...] += delta


@functools.partial(jax.jit, static_argnames=("interpret",))
def _run(x, W_router, w1, w2, w3, interpret=False):
    B, S, H = x.shape
    T = B * S
    xf = x.reshape(T, H)
    wr = jnp.zeros((EP, H), x.dtype).at[:E].set(W_router)
    dw = pl.pallas_call(
        _router_body,
        out_shape=jax.ShapeDtypeStruct((T, EP), jnp.float32),
        interpret=interpret,
    )(xf, wr)
    out = pl.pallas_call(
        _ffn_body,
        grid=(E, NF),
        in_specs=[
            pl.BlockSpec((T, EP), lambda e, f: (0, 0)),
            pl.BlockSpec((T, H), lambda e, f: (0, 0)),
            pl.BlockSpec((1, H, BF), lambda e, f: (e, 0, f)),
            pl.BlockSpec((1, H, BF), lambda e, f: (e, 0, f)),
            pl.BlockSpec((1, BF, H), lambda e, f: (e, f, 0)),
        ],
        out_specs=pl.BlockSpec((T, H), lambda e, f: (0, 0)),
        out_shape=jax.ShapeDtypeStruct((T, H), jnp.float32),
        compiler_params=pltpu.CompilerParams(
            dimension_semantics=("arbitrary", "arbitrary"),
        ),
        interpret=interpret,
    )(dw, xf, w1, w3, w2)
    return out.reshape(B, S, H)


def kernel(x, W_router, w1, w2, w3):
    return _run(x, W_router, w1, w2, w3)
